# baseline (speedup 1.0000x reference)
import functools

import jax
import jax.numpy as jnp
from jax import lax
from jax.experimental import pallas as pl
from jax.experimental.pallas import tpu as pltpu
from jax.experimental.pallas import tpu_sc as plsc

NC = 2
NS = 16
LANES = 16
NW = NC * NS
EPS = 1e-5
BLK = 1000
G = 64


def _sigmoid(z):
    return 1.0 / (1.0 + jnp.exp(-z))


def _edge_phase(kh, qh, vh, src, dst, zeros):
    _, N, Dh = kh.shape
    E = src.shape[0]
    per_t = E // NS
    C = 40
    chunks = per_t // C
    pairs = chunks // 2
    rem = chunks % 2
    rows_pt = 8 * (N // (8 * NS))
    tail_base = NS * rows_pt
    tail_rows = N - tail_base

    src3 = src.reshape(NS, chunks, C)
    dst3 = dst.reshape(NS, chunks, C)
    srco = jnp.stack([src3, src3 + N])
    dsto = jnp.stack([dst3, dst3 + N])
    k2 = kh.reshape(2 * N, Dh)
    q2 = qh.reshape(2 * N, Dh)
    v2 = vh.reshape(2 * N, Dh)

    mesh = plsc.VectorSubcoreMesh(core_axis_name="c", subcore_axis_name="s",
                                  num_cores=NC, num_subcores=NS)

    @functools.partial(
        pl.kernel,
        out_type=jax.ShapeDtypeStruct((NC, N, Dh), jnp.float32),
        mesh=mesh,
        compiler_params=pltpu.CompilerParams(use_tc_tiling_on_sc=False),
        scratch_types=[
            pltpu.VMEM((C,), jnp.int32),
            pltpu.VMEM((C,), jnp.int32),
            pltpu.VMEM((C,), jnp.int32),
            pltpu.VMEM((C,), jnp.int32),
            pltpu.VMEM((C,), jnp.int32),
            pltpu.VMEM((C,), jnp.int32),
            pltpu.VMEM((2, C, Dh), jnp.float32),
            pltpu.VMEM((2, C, Dh), jnp.float32),
            pltpu.VMEM((2, C, Dh), jnp.float32),
            pltpu.VMEM((2, C, Dh), jnp.float32),
            pltpu.VMEM_SHARED((N, Dh), jnp.float32),
            pltpu.SemaphoreType.DMA,
            pltpu.SemaphoreType.DMA,
            pltpu.SemaphoreType.DMA,
            pltpu.SemaphoreType.DMA,
        ],
    )
    def edge_kernel(k_hbm, q_hbm, v_hbm, srco_hbm, dsto_hbm, dst3_hbm, z_hbm,
                    out_hbm,
                    sig0, sig1, dig0, dig1, dis0, dis1,
                    kd, qs, vs, msg, acc,
                    gsem0, gsem1, ssem0, ssem1):
        cid = lax.axis_index("c")
        sid = lax.axis_index("s")
        r0 = sid * rows_pt
        sigs = (sig0, sig1)
        digs = (dig0, dig1)
        diss = (dis0, dis1)
        gsems = (gsem0, gsem1)
        ssems = (ssem0, ssem1)

        pltpu.sync_copy(z_hbm.at[pl.ds(r0, rows_pt)], acc.at[pl.ds(r0, rows_pt)])
        if tail_rows:
            @pl.when(sid == NS - 1)
            def _():
                pltpu.sync_copy(z_hbm.at[pl.ds(tail_base, tail_rows)],
                                acc.at[pl.ds(tail_base, tail_rows)])

        def load_gidx_and_fire(b, c):
            pltpu.sync_copy(srco_hbm.at[cid, sid, c], sigs[b])
            pltpu.sync_copy(dsto_hbm.at[cid, sid, c], digs[b])
            pltpu.async_copy(k_hbm.at[digs[b]], kd.at[b], gsems[b])
            pltpu.async_copy(q_hbm.at[sigs[b]], qs.at[b], gsems[b])
            pltpu.async_copy(v_hbm.at[sigs[b]], vs.at[b], gsems[b])

        def wait_gathers(b):
            pltpu.make_async_copy(k_hbm.at[digs[b]], kd.at[b], gsems[b]).wait()
            pltpu.make_async_copy(q_hbm.at[sigs[b]], qs.at[b], gsems[b]).wait()
            pltpu.make_async_copy(v_hbm.at[sigs[b]], vs.at[b], gsems[b]).wait()

        def wait_scatter(b):
            pltpu.make_async_copy(msg.at[b], acc.at[diss[b]], ssems[b]).wait()

        def compute(b):
            def row_body(r, carry):
                for cc in range(Dh // LANES):
                    sl = pl.ds(cc * LANES, LANES)
                    z = kd[b, r, sl] + qs[b, r, sl]
                    msg[b, r, sl] = _sigmoid(z) * vs[b, r, sl]
                return carry
            lax.fori_loop(0, C, row_body, 0)

        def fire_scatter(b, c):
            pltpu.sync_copy(dst3_hbm.at[sid, c], diss[b])
            pltpu.async_copy(msg.at[b], acc.at[diss[b]], ssems[b], add=True)

        plsc.subcore_barrier()

        load_gidx_and_fire(0, 0)
        load_gidx_and_fire(1, 1)

        def pair_body(i, carry):
            for b in range(2):
                c = 2 * i + b
                wait_gathers(b)

                @pl.when(i >= 1)
                def _():
                    wait_scatter(b)

                compute(b)
                fire_scatter(b, c)

                @pl.when(c + 2 < chunks)
                def _():
                    load_gidx_and_fire(b, c + 2)
            return carry

        lax.fori_loop(0, pairs, pair_body, 0)
        if rem:
            c = chunks - 1
            wait_gathers(0)
            wait_scatter(0)
            compute(0)
            fire_scatter(0, c)
        wait_scatter(0)
        wait_scatter(1)

        plsc.subcore_barrier()
        pltpu.sync_copy(acc.at[pl.ds(r0, rows_pt)],
                        out_hbm.at[cid, pl.ds(r0, rows_pt)])
        if tail_rows:
            @pl.when(sid == NS - 1)
            def _():
                pltpu.sync_copy(acc.at[pl.ds(tail_base, tail_rows)],
                                out_hbm.at[cid, pl.ds(tail_base, tail_rows)])

    return edge_kernel(k2, q2, v2, srco, dsto, dst3, zeros)


def _dense4(h, wall, ball):
    N, D = h.shape
    Dh = D // 2
    grid = N // BLK

    def kern(h_ref, w_ref, b_ref, k_ref, q_ref, v_ref, s_ref):
        y = jnp.dot(h_ref[...], w_ref[...],
                    preferred_element_type=jnp.float32) + b_ref[...]
        for o, t in ((0, k_ref), (1, q_ref), (2, v_ref)):
            t[0] = y[:, 2 * o * Dh:(2 * o + 1) * Dh]
            t[1] = y[:, (2 * o + 1) * Dh:(2 * o + 2) * Dh]
        s_ref[...] = y[:, 3 * D:4 * D]

    return pl.pallas_call(
        kern,
        grid=(grid,),
        in_specs=[pl.BlockSpec((BLK, D), lambda i: (i, 0)),
                  pl.BlockSpec((D, 4 * D), lambda i: (0, 0)),
                  pl.BlockSpec((1, 4 * D), lambda i: (0, 0))],
        out_specs=[pl.BlockSpec((2, BLK, Dh), lambda i: (0, i, 0))] * 3
        + [pl.BlockSpec((BLK, D), lambda i: (i, 0))],
        out_shape=[jax.ShapeDtypeStruct((2, N, Dh), jnp.float32)] * 3
        + [jax.ShapeDtypeStruct((N, D), jnp.float32)],
    )(h, wall, ball)


def _sig_stats(agg2, s):
    N, D = s.shape
    grid = N // BLK

    def kern(a_ref, s_ref, t_ref, ps_ref, pq_ref):
        agg = jnp.concatenate([a_ref[0], a_ref[1]], axis=1)
        t = _sigmoid(agg + s_ref[...])
        t_ref[...] = t
        ps_ref[0, 0, :] = jnp.sum(t, axis=0)
        pq_ref[0, 0, :] = jnp.sum(t * t, axis=0)

    return pl.pallas_call(
        kern,
        grid=(grid,),
        in_specs=[pl.BlockSpec((2, BLK, D // 2), lambda i: (0, i, 0)),
                  pl.BlockSpec((BLK, D), lambda i: (i, 0))],
        out_specs=[pl.BlockSpec((BLK, D), lambda i: (i, 0)),
                   pl.BlockSpec((1, 1, D), lambda i: (i, 0, 0)),
                   pl.BlockSpec((1, 1, D), lambda i: (i, 0, 0))],
        out_shape=[jax.ShapeDtypeStruct((N, D), jnp.float32),
                   jax.ShapeDtypeStruct((grid, 1, D), jnp.float32),
                   jax.ShapeDtypeStruct((grid, 1, D), jnp.float32)],
    )(agg2, s)


def _bn_dense4(t, ps, pq, gamma, beta, wall, ball):
    N, D = t.shape
    grid = N // BLK

    Dh = D // 2

    def kern(t_ref, ps_ref, pq_ref, g_ref, be_ref, w_ref, b_ref,
             h_ref, k_ref, q_ref, v_ref, s_ref):
        mu = jnp.sum(ps_ref[...], axis=0) / N
        var = jnp.sum(pq_ref[...], axis=0) / N - mu * mu
        scale = g_ref[...] * lax.rsqrt(var + EPS)
        h = (t_ref[...] - mu) * scale + be_ref[...]
        h_ref[...] = h
        y = jnp.dot(h, w_ref[...], preferred_element_type=jnp.float32) + b_ref[...]
        for o, t2 in ((0, k_ref), (1, q_ref), (2, v_ref)):
            t2[0] = y[:, 2 * o * Dh:(2 * o + 1) * Dh]
            t2[1] = y[:, (2 * o + 1) * Dh:(2 * o + 2) * Dh]
        s_ref[...] = y[:, 3 * D:4 * D]

    return pl.pallas_call(
        kern,
        grid=(grid,),
        in_specs=[pl.BlockSpec((BLK, D), lambda i: (i, 0)),
                  pl.BlockSpec((grid, 1, D), lambda i: (0, 0, 0)),
                  pl.BlockSpec((grid, 1, D), lambda i: (0, 0, 0)),
                  pl.BlockSpec((1, D), lambda i: (0, 0)),
                  pl.BlockSpec((1, D), lambda i: (0, 0)),
                  pl.BlockSpec((D, 4 * D), lambda i: (0, 0)),
                  pl.BlockSpec((1, 4 * D), lambda i: (0, 0))],
        out_specs=[pl.BlockSpec((BLK, D), lambda i: (i, 0))]
        + [pl.BlockSpec((2, BLK, Dh), lambda i: (0, i, 0))] * 3
        + [pl.BlockSpec((BLK, D), lambda i: (i, 0))],
        out_shape=[jax.ShapeDtypeStruct((N, D), jnp.float32)]
        + [jax.ShapeDtypeStruct((2, N, Dh), jnp.float32)] * 3
        + [jax.ShapeDtypeStruct((N, D), jnp.float32)],
    )(t, ps, pq, gamma, beta, wall, ball)


def _bn_pool(t1, ps, pq, gamma, beta, h0, batch3):
    N, D = t1.shape
    grid = N // BLK

    def kern(t_ref, ps_ref, pq_ref, g_ref, be_ref, h0_ref, bt_ref,
             s0_ref, s1_ref, c_ref):
        i = pl.program_id(0)
        mu = jnp.sum(ps_ref[...], axis=0) / N
        var = jnp.sum(pq_ref[...], axis=0) / N - mu * mu
        scale = g_ref[...] * lax.rsqrt(var + EPS)
        h1 = (t_ref[...] - mu) * scale + be_ref[...]
        gids = lax.broadcasted_iota(jnp.int32, (G, BLK), 0)
        oh = (gids == bt_ref[0]).astype(jnp.float32)
        part0 = jnp.dot(oh, h0_ref[...], preferred_element_type=jnp.float32)
        part1 = jnp.dot(oh, h1, preferred_element_type=jnp.float32)
        cnt = jnp.broadcast_to(jnp.sum(oh, axis=1, keepdims=True), (G, D))

        @pl.when(i == 0)
        def _():
            s0_ref[...] = jnp.zeros((G, D), jnp.float32)
            s1_ref[...] = jnp.zeros((G, D), jnp.float32)
            c_ref[...] = jnp.zeros((G, D), jnp.float32)

        s0_ref[...] += part0
        s1_ref[...] += part1
        c_ref[...] += cnt

    return pl.pallas_call(
        kern,
        grid=(grid,),
        in_specs=[pl.BlockSpec((BLK, D), lambda i: (i, 0)),
                  pl.BlockSpec((grid, 1, D), lambda i: (0, 0, 0)),
                  pl.BlockSpec((grid, 1, D), lambda i: (0, 0, 0)),
                  pl.BlockSpec((1, D), lambda i: (0, 0)),
                  pl.BlockSpec((1, D), lambda i: (0, 0)),
                  pl.BlockSpec((BLK, D), lambda i: (i, 0)),
                  pl.BlockSpec((1, 1, BLK), lambda i: (i, 0, 0))],
        out_specs=[pl.BlockSpec((G, D), lambda i: (0, 0))] * 3,
        out_shape=[jax.ShapeDtypeStruct((G, D), jnp.float32)] * 3,
    )(t1, ps, pq, gamma, beta, h0, batch3)


def _head(s0, s1, cnt, w0, b0, g0, be0, w1, b1, g1, be1, wc, bc):
    D = s0.shape[1]

    def bn(xv, g, b):
        mu = jnp.mean(xv, axis=0, keepdims=True)
        var = jnp.mean((xv - mu) * (xv - mu), axis=0, keepdims=True)
        return (xv - mu) * lax.rsqrt(var + EPS) * g + b

    def kern(s0_ref, s1_ref, c_ref, w0_ref, b0_ref, g0_ref, be0_ref,
             w1_ref, b1_ref, g1_ref, be1_ref, wc_ref, bc_ref, o_ref):
        c = jnp.maximum(c_ref[...], 1.0)
        feat = jnp.concatenate(
            [s0_ref[...] / c, s1_ref[...] / c, s0_ref[...], s1_ref[...]],
            axis=1)
        xv = jnp.dot(feat, w0_ref[...], preferred_element_type=jnp.float32) + b0_ref[...]
        xv = bn(_sigmoid(xv), g0_ref[...], be0_ref[...])
        xv = jnp.dot(xv, w1_ref[...], preferred_element_type=jnp.float32) + b1_ref[...]
        xv = bn(_sigmoid(xv), g1_ref[...], be1_ref[...])
        o_ref[...] = jnp.dot(xv, wc_ref[...], preferred_element_type=jnp.float32) + bc_ref[...]

    return pl.pallas_call(
        kern,
        out_shape=jax.ShapeDtypeStruct((G, D), jnp.float32),
    )(s0, s1, cnt, w0, b0, g0, be0, w1, b1, g1, be1, wc, bc)


def kernel(x, edge_index, batch, params):
    N, D = x.shape
    src = edge_index[0]
    dst = edge_index[1]
    grid = N // BLK
    zeros = jnp.zeros((N, D // 2), jnp.float32)
    batch3 = batch.reshape(grid, 1, BLK)

    def wall(i):
        w = jnp.concatenate([params['conv%d_%s_W' % (i, nm)]
                             for nm in ('key', 'query', 'value', 'skip')], axis=1)
        b = jnp.concatenate([params['conv%d_%s_b' % (i, nm)]
                             for nm in ('key', 'query', 'value', 'skip')]).reshape(1, 4 * D)
        return w, b

    w0c, b0c = wall(0)
    w1c, b1c = wall(1)
    g0 = params['bn0_gamma'].reshape(1, D)
    be0 = params['bn0_beta'].reshape(1, D)
    g1 = params['bn1_gamma'].reshape(1, D)
    be1 = params['bn1_beta'].reshape(1, D)

    k0, q0, v0, s0 = _dense4(x, w0c, b0c)
    agg0 = _edge_phase(k0, q0, v0, src, dst, zeros)
    t0, ps0, pq0 = _sig_stats(agg0, s0)
    h0, k1, q1, v1, s1 = _bn_dense4(t0, ps0, pq0, g0, be0, w1c, b1c)
    agg1 = _edge_phase(k1, q1, v1, src, dst, zeros)
    t1, ps1, pq1 = _sig_stats(agg1, s1)
    sums0, sums1, counts = _bn_pool(t1, ps1, pq1, g1, be1, h0, batch3)

    wc = jnp.pad(params['cls_W'], ((0, 0), (0, D - params['cls_W'].shape[1])))
    bc = jnp.pad(params['cls_b'], (0, D - params['cls_b'].shape[0])).reshape(1, D)
    out = _head(sums0, sums1, counts,
                params['hl0_W'], params['hl0_b'].reshape(1, -1),
                params['hbn0_gamma'].reshape(1, -1), params['hbn0_beta'].reshape(1, -1),
                params['hl1_W'], params['hl1_b'].reshape(1, -1),
                params['hbn1_gamma'].reshape(1, -1), params['hbn1_beta'].reshape(1, -1),
                wc, bc)
    return out[:, :params['cls_W'].shape[1]]


# speedup vs baseline: 2.2649x; 2.2649x over previous
import functools

import jax
import jax.numpy as jnp
from jax import lax
from jax.experimental import pallas as pl
from jax.experimental.pallas import tpu as pltpu
from jax.experimental.pallas import tpu_sc as plsc

NC = 2
NS = 16
LANES = 16
NW = NC * NS
EPS = 1e-5
BLK = 1000
G = 64


def _sigmoid(z):
    return 1.0 / (1.0 + jnp.exp(-z))


def _edge_phase(k, q, v, src, dst, zeros):
    N, D = k.shape
    E = src.shape[0]
    per_w = E // NW
    C = 40
    chunks = per_w // C
    pairs = chunks // 2
    rem = chunks % 2
    assert chunks >= 4
    rows_pt = 8 * (N // (8 * NS))
    tail_base = NS * rows_pt
    tail_rows = N - tail_base

    mesh = plsc.VectorSubcoreMesh(core_axis_name="c", subcore_axis_name="s",
                                  num_cores=NC, num_subcores=NS)

    @functools.partial(
        pl.kernel,
        out_type=jax.ShapeDtypeStruct((NC, N, D), jnp.float32),
        mesh=mesh,
        scratch_types=[
            pltpu.VMEM((C,), jnp.int32),
            pltpu.VMEM((C,), jnp.int32),
            pltpu.VMEM((C,), jnp.int32),
            pltpu.VMEM((C,), jnp.int32),
            pltpu.VMEM((C,), jnp.int32),
            pltpu.VMEM((C,), jnp.int32),
            pltpu.VMEM((C,), jnp.int32),
            pltpu.VMEM((C,), jnp.int32),
            pltpu.VMEM((C,), jnp.int32),
            pltpu.VMEM((2, C, D), jnp.float32),
            pltpu.VMEM((2, C, D), jnp.float32),
            pltpu.VMEM((2, C, D), jnp.float32),
            pltpu.VMEM_SHARED((N, D), jnp.float32),
            pltpu.SemaphoreType.DMA,
            pltpu.SemaphoreType.DMA,
            pltpu.SemaphoreType.DMA,
            pltpu.SemaphoreType.DMA,
            pltpu.SemaphoreType.DMA,
            pltpu.SemaphoreType.DMA,
        ],
    )
    def edge_kernel(k_hbm, q_hbm, v_hbm, src_hbm, dst_hbm, z_hbm, out_hbm,
                    sig0, sig1, sig2, sig3, dig0, dig1, dig2, dig3, dis,
                    kd, qs, vs, acc,
                    isem0, isem1, isem2, isem3, gsem0, gsem1):
        cid = lax.axis_index("c")
        sid = lax.axis_index("s")
        wid = sid * NC + cid
        base0 = wid * per_w
        r0 = sid * rows_pt
        sigs = (sig0, sig1, sig2, sig3)
        digs = (dig0, dig1, dig2, dig3)
        isems = (isem0, isem1, isem2, isem3)
        gsems = (gsem0, gsem1)

        pltpu.sync_copy(z_hbm.at[pl.ds(r0, rows_pt)], acc.at[pl.ds(r0, rows_pt)])
        if tail_rows:
            @pl.when(sid == NS - 1)
            def _():
                pltpu.sync_copy(z_hbm.at[pl.ds(tail_base, tail_rows)],
                                acc.at[pl.ds(tail_base, tail_rows)])

        def fire_idx(j, c):
            pltpu.async_copy(src_hbm.at[pl.ds(base0 + c * C, C)], sigs[j],
                             isems[j])
            pltpu.async_copy(dst_hbm.at[pl.ds(base0 + c * C, C)], digs[j],
                             isems[j])

        def wait_idx(j):
            pltpu.make_async_copy(src_hbm.at[pl.ds(base0, C)], sigs[j],
                                  isems[j]).wait()
            pltpu.make_async_copy(dst_hbm.at[pl.ds(base0, C)], digs[j],
                                  isems[j]).wait()

        def fire_gathers(b, j):
            pltpu.async_copy(k_hbm.at[digs[j]], kd.at[b], gsems[b])
            pltpu.async_copy(q_hbm.at[sigs[j]], qs.at[b], gsems[b])
            pltpu.async_copy(v_hbm.at[sigs[j]], vs.at[b], gsems[b])

        def wait_gathers(b, j):
            pltpu.make_async_copy(k_hbm.at[digs[j]], kd.at[b], gsems[b]).wait()
            pltpu.make_async_copy(q_hbm.at[sigs[j]], qs.at[b], gsems[b]).wait()
            pltpu.make_async_copy(v_hbm.at[sigs[j]], vs.at[b], gsems[b]).wait()

        def compute(b):
            def row_body(r, carry):
                for cc in range(D // LANES):
                    sl = pl.ds(cc * LANES, LANES)
                    z = kd[b, r, sl] + qs[b, r, sl]
                    vs[b, r, sl] = _sigmoid(z) * vs[b, r, sl]
                return carry
            lax.fori_loop(0, C, row_body, 0)

        def scatter(b, c):
            pltpu.sync_copy(dst_hbm.at[pl.ds(base0 + c * C, C)], dis)
            pltpu.sync_copy(vs.at[b], acc.at[dis], add=True)

        plsc.subcore_barrier()

        for j in range(4):
            fire_idx(j, j)
        wait_idx(0)
        fire_gathers(0, 0)
        wait_idx(1)
        fire_gathers(1, 1)

        def body(c, b, j, jn):
            wait_gathers(b, j)
            compute(b)
            scatter(b, c)

            @pl.when(c + 2 < chunks)
            def _():
                wait_idx(jn)
                fire_gathers(b, jn)

            @pl.when(c + 4 < chunks)
            def _():
                fire_idx(j, c + 4)

        quads = chunks // 4
        qrem = chunks % 4

        def quad_body(iq, carry):
            c0 = 4 * iq
            for u in range(4):
                body(c0 + u, u % 2, u, (u + 2) % 4)
            return carry

        lax.fori_loop(0, quads, quad_body, 0)
        for u in range(qrem):
            body(quads * 4 + u, u % 2, u, (u + 2) % 4)

        plsc.subcore_barrier()
        pltpu.sync_copy(acc.at[pl.ds(r0, rows_pt)],
                        out_hbm.at[cid, pl.ds(r0, rows_pt)])
        if tail_rows:
            @pl.when(sid == NS - 1)
            def _():
                pltpu.sync_copy(acc.at[pl.ds(tail_base, tail_rows)],
                                out_hbm.at[cid, pl.ds(tail_base, tail_rows)])

    return edge_kernel(k, q, v, src, dst, zeros)


def _dense4(h, wall, ball):
    N, D = h.shape
    grid = N // BLK

    def kern(h_ref, w_ref, b_ref, k_ref, q_ref, v_ref, s_ref):
        y = jnp.dot(h_ref[...], w_ref[...],
                    preferred_element_type=jnp.float32) + b_ref[...]
        k_ref[...] = y[:, 0 * D:1 * D]
        q_ref[...] = y[:, 1 * D:2 * D]
        v_ref[...] = y[:, 2 * D:3 * D]
        s_ref[...] = y[:, 3 * D:4 * D]

    return pl.pallas_call(
        kern,
        grid=(grid,),
        in_specs=[pl.BlockSpec((BLK, D), lambda i: (i, 0)),
                  pl.BlockSpec((D, 4 * D), lambda i: (0, 0)),
                  pl.BlockSpec((1, 4 * D), lambda i: (0, 0))],
        out_specs=[pl.BlockSpec((BLK, D), lambda i: (i, 0))] * 4,
        out_shape=[jax.ShapeDtypeStruct((N, D), jnp.float32)] * 4,
    )(h, wall, ball)


def _sig_stats(agg2, s):
    N, D = s.shape
    grid = N // BLK

    def kern(a_ref, s_ref, t_ref, ps_ref, pq_ref):
        t = _sigmoid(a_ref[0] + a_ref[1] + s_ref[...])
        t_ref[...] = t
        ps_ref[0, 0, :] = jnp.sum(t, axis=0)
        pq_ref[0, 0, :] = jnp.sum(t * t, axis=0)

    return pl.pallas_call(
        kern,
        grid=(grid,),
        in_specs=[pl.BlockSpec((2, BLK, D), lambda i: (0, i, 0)),
                  pl.BlockSpec((BLK, D), lambda i: (i, 0))],
        out_specs=[pl.BlockSpec((BLK, D), lambda i: (i, 0)),
                   pl.BlockSpec((1, 1, D), lambda i: (i, 0, 0)),
                   pl.BlockSpec((1, 1, D), lambda i: (i, 0, 0))],
        out_shape=[jax.ShapeDtypeStruct((N, D), jnp.float32),
                   jax.ShapeDtypeStruct((grid, 1, D), jnp.float32),
                   jax.ShapeDtypeStruct((grid, 1, D), jnp.float32)],
    )(agg2, s)


def _bn_dense4(t, ps, pq, gamma, beta, wall, ball):
    N, D = t.shape
    grid = N // BLK

    def kern(t_ref, ps_ref, pq_ref, g_ref, be_ref, w_ref, b_ref,
             h_ref, k_ref, q_ref, v_ref, s_ref):
        mu = jnp.sum(ps_ref[...], axis=0) / N
        var = jnp.sum(pq_ref[...], axis=0) / N - mu * mu
        scale = g_ref[...] * lax.rsqrt(var + EPS)
        h = (t_ref[...] - mu) * scale + be_ref[...]
        h_ref[...] = h
        y = jnp.dot(h, w_ref[...], preferred_element_type=jnp.float32) + b_ref[...]
        k_ref[...] = y[:, 0 * D:1 * D]
        q_ref[...] = y[:, 1 * D:2 * D]
        v_ref[...] = y[:, 2 * D:3 * D]
        s_ref[...] = y[:, 3 * D:4 * D]

    return pl.pallas_call(
        kern,
        grid=(grid,),
        in_specs=[pl.BlockSpec((BLK, D), lambda i: (i, 0)),
                  pl.BlockSpec((grid, 1, D), lambda i: (0, 0, 0)),
                  pl.BlockSpec((grid, 1, D), lambda i: (0, 0, 0)),
                  pl.BlockSpec((1, D), lambda i: (0, 0)),
                  pl.BlockSpec((1, D), lambda i: (0, 0)),
                  pl.BlockSpec((D, 4 * D), lambda i: (0, 0)),
                  pl.BlockSpec((1, 4 * D), lambda i: (0, 0))],
        out_specs=[pl.BlockSpec((BLK, D), lambda i: (i, 0))] * 5,
        out_shape=[jax.ShapeDtypeStruct((N, D), jnp.float32)] * 5,
    )(t, ps, pq, gamma, beta, wall, ball)


def _bn_pool(t1, ps, pq, gamma, beta, h0, batch3):
    N, D = t1.shape
    grid = N // BLK

    def kern(t_ref, ps_ref, pq_ref, g_ref, be_ref, h0_ref, bt_ref,
             s0_ref, s1_ref, c_ref):
        i = pl.program_id(0)
        mu = jnp.sum(ps_ref[...], axis=0) / N
        var = jnp.sum(pq_ref[...], axis=0) / N - mu * mu
        scale = g_ref[...] * lax.rsqrt(var + EPS)
        h1 = (t_ref[...] - mu) * scale + be_ref[...]
        gids = lax.broadcasted_iota(jnp.int32, (G, BLK), 0)
        oh = (gids == bt_ref[0]).astype(jnp.float32)
        part0 = jnp.dot(oh, h0_ref[...], preferred_element_type=jnp.float32)
        part1 = jnp.dot(oh, h1, preferred_element_type=jnp.float32)
        cnt = jnp.broadcast_to(jnp.sum(oh, axis=1, keepdims=True), (G, D))

        @pl.when(i == 0)
        def _():
            s0_ref[...] = jnp.zeros((G, D), jnp.float32)
            s1_ref[...] = jnp.zeros((G, D), jnp.float32)
            c_ref[...] = jnp.zeros((G, D), jnp.float32)

        s0_ref[...] += part0
        s1_ref[...] += part1
        c_ref[...] += cnt

    return pl.pallas_call(
        kern,
        grid=(grid,),
        in_specs=[pl.BlockSpec((BLK, D), lambda i: (i, 0)),
                  pl.BlockSpec((grid, 1, D), lambda i: (0, 0, 0)),
                  pl.BlockSpec((grid, 1, D), lambda i: (0, 0, 0)),
                  pl.BlockSpec((1, D), lambda i: (0, 0)),
                  pl.BlockSpec((1, D), lambda i: (0, 0)),
                  pl.BlockSpec((BLK, D), lambda i: (i, 0)),
                  pl.BlockSpec((1, 1, BLK), lambda i: (i, 0, 0))],
        out_specs=[pl.BlockSpec((G, D), lambda i: (0, 0))] * 3,
        out_shape=[jax.ShapeDtypeStruct((G, D), jnp.float32)] * 3,
    )(t1, ps, pq, gamma, beta, h0, batch3)


def _head(s0, s1, cnt, w0, b0, g0, be0, w1, b1, g1, be1, wc, bc):
    def bn(xv, g, b):
        mu = jnp.mean(xv, axis=0, keepdims=True)
        var = jnp.mean((xv - mu) * (xv - mu), axis=0, keepdims=True)
        return (xv - mu) * lax.rsqrt(var + EPS) * g + b

    def kern(s0_ref, s1_ref, c_ref, w0_ref, b0_ref, g0_ref, be0_ref,
             w1_ref, b1_ref, g1_ref, be1_ref, wc_ref, bc_ref, o_ref):
        c = jnp.maximum(c_ref[...], 1.0)
        feat = jnp.concatenate(
            [s0_ref[...] / c, s1_ref[...] / c, s0_ref[...], s1_ref[...]],
            axis=1)
        xv = jnp.dot(feat, w0_ref[...], preferred_element_type=jnp.float32) + b0_ref[...]
        xv = bn(_sigmoid(xv), g0_ref[...], be0_ref[...])
        xv = jnp.dot(xv, w1_ref[...], preferred_element_type=jnp.float32) + b1_ref[...]
        xv = bn(_sigmoid(xv), g1_ref[...], be1_ref[...])
        o_ref[...] = jnp.dot(xv, wc_ref[...], preferred_element_type=jnp.float32) + bc_ref[...]

    D = s0.shape[1]
    return pl.pallas_call(
        kern,
        out_shape=jax.ShapeDtypeStruct((G, D), jnp.float32),
    )(s0, s1, cnt, w0, b0, g0, be0, w1, b1, g1, be1, wc, bc)


def kernel(x, edge_index, batch, params):
    N, D = x.shape
    src = edge_index[0]
    dst = edge_index[1]
    grid = N // BLK
    zeros = jnp.zeros((N, D), jnp.float32)
    batch3 = batch.reshape(grid, 1, BLK)

    def wall(i):
        w = jnp.concatenate([params['conv%d_%s_W' % (i, nm)]
                             for nm in ('key', 'query', 'value', 'skip')], axis=1)
        b = jnp.concatenate([params['conv%d_%s_b' % (i, nm)]
                             for nm in ('key', 'query', 'value', 'skip')]).reshape(1, 4 * D)
        return w, b

    w0c, b0c = wall(0)
    w1c, b1c = wall(1)
    g0 = params['bn0_gamma'].reshape(1, D)
    be0 = params['bn0_beta'].reshape(1, D)
    g1 = params['bn1_gamma'].reshape(1, D)
    be1 = params['bn1_beta'].reshape(1, D)

    k0, q0, v0, s0 = _dense4(x, w0c, b0c)
    agg0 = _edge_phase(k0, q0, v0, src, dst, zeros)
    t0, ps0, pq0 = _sig_stats(agg0, s0)
    h0, k1, q1, v1, s1 = _bn_dense4(t0, ps0, pq0, g0, be0, w1c, b1c)
    agg1 = _edge_phase(k1, q1, v1, src, dst, zeros)
    t1, ps1, pq1 = _sig_stats(agg1, s1)
    sums0, sums1, counts = _bn_pool(t1, ps1, pq1, g1, be1, h0, batch3)

    wc = jnp.pad(params['cls_W'], ((0, 0), (0, D - params['cls_W'].shape[1])))
    bc = jnp.pad(params['cls_b'], (0, D - params['cls_b'].shape[0])).reshape(1, D)
    out = _head(sums0, sums1, counts,
                params['hl0_W'], params['hl0_b'].reshape(1, -1),
                params['hbn0_gamma'].reshape(1, -1), params['hbn0_beta'].reshape(1, -1),
                params['hl1_W'], params['hl1_b'].reshape(1, -1),
                params['hbn1_gamma'].reshape(1, -1), params['hbn1_beta'].reshape(1, -1),
                wc, bc)
    return out[:, :params['cls_W'].shape[1]]


# speedup vs baseline: 2.6518x; 1.1708x over previous
import functools

import jax
import jax.numpy as jnp
from jax import lax
from jax.experimental import pallas as pl
from jax.experimental.pallas import tpu as pltpu
from jax.experimental.pallas import tpu_sc as plsc

NC = 2
NS = 16
LANES = 16
NW = NC * NS
EPS = 1e-5
BLK = 1000
G = 64


def _sigmoid(z):
    return 1.0 / (1.0 + jnp.exp(-z))


def _edge_phase(k, q, v, src, dst, zeros):
    N, D = k.shape
    E = src.shape[0]
    per_w = E // NW
    C = 40
    chunks = per_w // C
    pairs = chunks // 2
    rem = chunks % 2
    assert chunks >= 4
    rows_pt = 8 * (N // (8 * NS))
    tail_base = NS * rows_pt
    tail_rows = N - tail_base

    mesh = plsc.VectorSubcoreMesh(core_axis_name="c", subcore_axis_name="s",
                                  num_cores=NC, num_subcores=NS)

    @functools.partial(
        pl.kernel,
        out_type=jax.ShapeDtypeStruct((NC, N, D), jnp.float32),
        mesh=mesh,
        scratch_types=[
            pltpu.VMEM((C,), jnp.int32),
            pltpu.VMEM((C,), jnp.int32),
            pltpu.VMEM((C,), jnp.int32),
            pltpu.VMEM((C,), jnp.int32),
            pltpu.VMEM((C,), jnp.int32),
            pltpu.VMEM((C,), jnp.int32),
            pltpu.VMEM((C,), jnp.int32),
            pltpu.VMEM((C,), jnp.int32),
            pltpu.VMEM((2, C, D), jnp.float32),
            pltpu.VMEM((2, C, D), jnp.float32),
            pltpu.VMEM((2, C, D), jnp.float32),
            pltpu.VMEM_SHARED((N, D), jnp.float32),
            pltpu.SemaphoreType.DMA,
            pltpu.SemaphoreType.DMA,
            pltpu.SemaphoreType.DMA,
            pltpu.SemaphoreType.DMA,
            pltpu.SemaphoreType.DMA,
            pltpu.SemaphoreType.DMA,
        ],
    )
    def edge_kernel(k_hbm, q_hbm, v_hbm, src_hbm, dst_hbm, z_hbm, out_hbm,
                    sig0, sig1, sig2, sig3, dig0, dig1, dig2, dig3,
                    kd, qs, vs, acc,
                    isem0, isem1, isem2, isem3, gsem0, gsem1):
        cid = lax.axis_index("c")
        sid = lax.axis_index("s")
        wid = sid * NC + cid
        base0 = wid * per_w
        r0 = sid * rows_pt
        sigs = (sig0, sig1, sig2, sig3)
        digs = (dig0, dig1, dig2, dig3)
        isems = (isem0, isem1, isem2, isem3)
        gsems = (gsem0, gsem1)

        pltpu.sync_copy(z_hbm.at[pl.ds(r0, rows_pt)], acc.at[pl.ds(r0, rows_pt)])
        if tail_rows:
            @pl.when(sid == NS - 1)
            def _():
                pltpu.sync_copy(z_hbm.at[pl.ds(tail_base, tail_rows)],
                                acc.at[pl.ds(tail_base, tail_rows)])

        def fire_idx(j, c):
            pltpu.async_copy(src_hbm.at[pl.ds(base0 + c * C, C)], sigs[j],
                             isems[j])
            pltpu.async_copy(dst_hbm.at[pl.ds(base0 + c * C, C)], digs[j],
                             isems[j])

        def wait_idx(j):
            pltpu.make_async_copy(src_hbm.at[pl.ds(base0, C)], sigs[j],
                                  isems[j]).wait()
            pltpu.make_async_copy(dst_hbm.at[pl.ds(base0, C)], digs[j],
                                  isems[j]).wait()

        def fire_gathers(b, j):
            pltpu.async_copy(k_hbm.at[digs[j]], kd.at[b], gsems[b])
            pltpu.async_copy(q_hbm.at[sigs[j]], qs.at[b], gsems[b])
            pltpu.async_copy(v_hbm.at[sigs[j]], vs.at[b], gsems[b])

        def wait_gathers(b, j):
            pltpu.make_async_copy(k_hbm.at[digs[j]], kd.at[b], gsems[b]).wait()
            pltpu.make_async_copy(q_hbm.at[sigs[j]], qs.at[b], gsems[b]).wait()
            pltpu.make_async_copy(v_hbm.at[sigs[j]], vs.at[b], gsems[b]).wait()

        def compute(b):
            def row_body(r, carry):
                for cc in range(D // LANES):
                    sl = pl.ds(cc * LANES, LANES)
                    z = kd[b, r, sl] + qs[b, r, sl]
                    vs[b, r, sl] = _sigmoid(z) * vs[b, r, sl]
                return carry
            lax.fori_loop(0, C, row_body, 0)

        def scatter(b, j):
            pltpu.sync_copy(vs.at[b], acc.at[digs[j]], add=True)

        plsc.subcore_barrier()

        for j in range(4):
            fire_idx(j, j)
        wait_idx(0)
        fire_gathers(0, 0)
        wait_idx(1)
        fire_gathers(1, 1)

        def body(c, b, j, jn):
            wait_gathers(b, j)
            compute(b)
            scatter(b, j)

            @pl.when(c + 2 < chunks)
            def _():
                wait_idx(jn)
                fire_gathers(b, jn)

            @pl.when(c + 4 < chunks)
            def _():
                fire_idx(j, c + 4)

        quads = chunks // 4
        qrem = chunks % 4

        def quad_body(iq, carry):
            c0 = 4 * iq
            for u in range(4):
                body(c0 + u, u % 2, u, (u + 2) % 4)
            return carry

        lax.fori_loop(0, quads, quad_body, 0)
        for u in range(qrem):
            body(quads * 4 + u, u % 2, u, (u + 2) % 4)

        plsc.subcore_barrier()
        pltpu.sync_copy(acc.at[pl.ds(r0, rows_pt)],
                        out_hbm.at[cid, pl.ds(r0, rows_pt)])
        if tail_rows:
            @pl.when(sid == NS - 1)
            def _():
                pltpu.sync_copy(acc.at[pl.ds(tail_base, tail_rows)],
                                out_hbm.at[cid, pl.ds(tail_base, tail_rows)])

    return edge_kernel(k, q, v, src, dst, zeros)


def _dense4(h, wall, ball):
    N, D = h.shape
    grid = N // BLK

    def kern(h_ref, w_ref, b_ref, k_ref, q_ref, v_ref, s_ref):
        y = jnp.dot(h_ref[...], w_ref[...],
                    preferred_element_type=jnp.float32) + b_ref[...]
        k_ref[...] = y[:, 0 * D:1 * D]
        q_ref[...] = y[:, 1 * D:2 * D]
        v_ref[...] = y[:, 2 * D:3 * D]
        s_ref[...] = y[:, 3 * D:4 * D]

    return pl.pallas_call(
        kern,
        grid=(grid,),
        in_specs=[pl.BlockSpec((BLK, D), lambda i: (i, 0)),
                  pl.BlockSpec((D, 4 * D), lambda i: (0, 0)),
                  pl.BlockSpec((1, 4 * D), lambda i: (0, 0))],
        out_specs=[pl.BlockSpec((BLK, D), lambda i: (i, 0))] * 4,
        out_shape=[jax.ShapeDtypeStruct((N, D), jnp.float32)] * 4,
    )(h, wall, ball)


def _sig_stats(agg2, s):
    N, D = s.shape
    grid = N // BLK

    def kern(a_ref, s_ref, t_ref, ps_ref, pq_ref):
        t = _sigmoid(a_ref[0] + a_ref[1] + s_ref[...])
        t_ref[...] = t
        ps_ref[0, 0, :] = jnp.sum(t, axis=0)
        pq_ref[0, 0, :] = jnp.sum(t * t, axis=0)

    return pl.pallas_call(
        kern,
        grid=(grid,),
        in_specs=[pl.BlockSpec((2, BLK, D), lambda i: (0, i, 0)),
                  pl.BlockSpec((BLK, D), lambda i: (i, 0))],
        out_specs=[pl.BlockSpec((BLK, D), lambda i: (i, 0)),
                   pl.BlockSpec((1, 1, D), lambda i: (i, 0, 0)),
                   pl.BlockSpec((1, 1, D), lambda i: (i, 0, 0))],
        out_shape=[jax.ShapeDtypeStruct((N, D), jnp.float32),
                   jax.ShapeDtypeStruct((grid, 1, D), jnp.float32),
                   jax.ShapeDtypeStruct((grid, 1, D), jnp.float32)],
    )(agg2, s)


def _bn_dense4(t, ps, pq, gamma, beta, wall, ball):
    N, D = t.shape
    grid = N // BLK

    def kern(t_ref, ps_ref, pq_ref, g_ref, be_ref, w_ref, b_ref,
             h_ref, k_ref, q_ref, v_ref, s_ref):
        mu = jnp.sum(ps_ref[...], axis=0) / N
        var = jnp.sum(pq_ref[...], axis=0) / N - mu * mu
        scale = g_ref[...] * lax.rsqrt(var + EPS)
        h = (t_ref[...] - mu) * scale + be_ref[...]
        h_ref[...] = h
        y = jnp.dot(h, w_ref[...], preferred_element_type=jnp.float32) + b_ref[...]
        k_ref[...] = y[:, 0 * D:1 * D]
        q_ref[...] = y[:, 1 * D:2 * D]
        v_ref[...] = y[:, 2 * D:3 * D]
        s_ref[...] = y[:, 3 * D:4 * D]

    return pl.pallas_call(
        kern,
        grid=(grid,),
        in_specs=[pl.BlockSpec((BLK, D), lambda i: (i, 0)),
                  pl.BlockSpec((grid, 1, D), lambda i: (0, 0, 0)),
                  pl.BlockSpec((grid, 1, D), lambda i: (0, 0, 0)),
                  pl.BlockSpec((1, D), lambda i: (0, 0)),
                  pl.BlockSpec((1, D), lambda i: (0, 0)),
                  pl.BlockSpec((D, 4 * D), lambda i: (0, 0)),
                  pl.BlockSpec((1, 4 * D), lambda i: (0, 0))],
        out_specs=[pl.BlockSpec((BLK, D), lambda i: (i, 0))] * 5,
        out_shape=[jax.ShapeDtypeStruct((N, D), jnp.float32)] * 5,
    )(t, ps, pq, gamma, beta, wall, ball)


def _bn_pool(t1, ps, pq, gamma, beta, h0, batch3):
    N, D = t1.shape
    grid = N // BLK

    def kern(t_ref, ps_ref, pq_ref, g_ref, be_ref, h0_ref, bt_ref,
             s0_ref, s1_ref, c_ref):
        i = pl.program_id(0)
        mu = jnp.sum(ps_ref[...], axis=0) / N
        var = jnp.sum(pq_ref[...], axis=0) / N - mu * mu
        scale = g_ref[...] * lax.rsqrt(var + EPS)
        h1 = (t_ref[...] - mu) * scale + be_ref[...]
        gids = lax.broadcasted_iota(jnp.int32, (G, BLK), 0)
        oh = (gids == bt_ref[0]).astype(jnp.float32)
        part0 = jnp.dot(oh, h0_ref[...], preferred_element_type=jnp.float32)
        part1 = jnp.dot(oh, h1, preferred_element_type=jnp.float32)
        cnt = jnp.broadcast_to(jnp.sum(oh, axis=1, keepdims=True), (G, D))

        @pl.when(i == 0)
        def _():
            s0_ref[...] = jnp.zeros((G, D), jnp.float32)
            s1_ref[...] = jnp.zeros((G, D), jnp.float32)
            c_ref[...] = jnp.zeros((G, D), jnp.float32)

        s0_ref[...] += part0
        s1_ref[...] += part1
        c_ref[...] += cnt

    return pl.pallas_call(
        kern,
        grid=(grid,),
        in_specs=[pl.BlockSpec((BLK, D), lambda i: (i, 0)),
                  pl.BlockSpec((grid, 1, D), lambda i: (0, 0, 0)),
                  pl.BlockSpec((grid, 1, D), lambda i: (0, 0, 0)),
                  pl.BlockSpec((1, D), lambda i: (0, 0)),
                  pl.BlockSpec((1, D), lambda i: (0, 0)),
                  pl.BlockSpec((BLK, D), lambda i: (i, 0)),
                  pl.BlockSpec((1, 1, BLK), lambda i: (i, 0, 0))],
        out_specs=[pl.BlockSpec((G, D), lambda i: (0, 0))] * 3,
        out_shape=[jax.ShapeDtypeStruct((G, D), jnp.float32)] * 3,
    )(t1, ps, pq, gamma, beta, h0, batch3)


def _head(s0, s1, cnt, w0, b0, g0, be0, w1, b1, g1, be1, wc, bc):
    def bn(xv, g, b):
        mu = jnp.mean(xv, axis=0, keepdims=True)
        var = jnp.mean((xv - mu) * (xv - mu), axis=0, keepdims=True)
        return (xv - mu) * lax.rsqrt(var + EPS) * g + b

    def kern(s0_ref, s1_ref, c_ref, w0_ref, b0_ref, g0_ref, be0_ref,
             w1_ref, b1_ref, g1_ref, be1_ref, wc_ref, bc_ref, o_ref):
        c = jnp.maximum(c_ref[...], 1.0)
        feat = jnp.concatenate(
            [s0_ref[...] / c, s1_ref[...] / c, s0_ref[...], s1_ref[...]],
            axis=1)
        xv = jnp.dot(feat, w0_ref[...], preferred_element_type=jnp.float32) + b0_ref[...]
        xv = bn(_sigmoid(xv), g0_ref[...], be0_ref[...])
        xv = jnp.dot(xv, w1_ref[...], preferred_element_type=jnp.float32) + b1_ref[...]
        xv = bn(_sigmoid(xv), g1_ref[...], be1_ref[...])
        o_ref[...] = jnp.dot(xv, wc_ref[...], preferred_element_type=jnp.float32) + bc_ref[...]

    D = s0.shape[1]
    return pl.pallas_call(
        kern,
        out_shape=jax.ShapeDtypeStruct((G, D), jnp.float32),
    )(s0, s1, cnt, w0, b0, g0, be0, w1, b1, g1, be1, wc, bc)


def kernel(x, edge_index, batch, params):
    N, D = x.shape
    src = edge_index[0]
    dst = edge_index[1]
    grid = N // BLK
    zeros = jnp.zeros((N, D), jnp.float32)
    batch3 = batch.reshape(grid, 1, BLK)

    def wall(i):
        w = jnp.concatenate([params['conv%d_%s_W' % (i, nm)]
                             for nm in ('key', 'query', 'value', 'skip')], axis=1)
        b = jnp.concatenate([params['conv%d_%s_b' % (i, nm)]
                             for nm in ('key', 'query', 'value', 'skip')]).reshape(1, 4 * D)
        return w, b

    w0c, b0c = wall(0)
    w1c, b1c = wall(1)
    g0 = params['bn0_gamma'].reshape(1, D)
    be0 = params['bn0_beta'].reshape(1, D)
    g1 = params['bn1_gamma'].reshape(1, D)
    be1 = params['bn1_beta'].reshape(1, D)

    k0, q0, v0, s0 = _dense4(x, w0c, b0c)
    agg0 = _edge_phase(k0, q0, v0, src, dst, zeros)
    t0, ps0, pq0 = _sig_stats(agg0, s0)
    h0, k1, q1, v1, s1 = _bn_dense4(t0, ps0, pq0, g0, be0, w1c, b1c)
    agg1 = _edge_phase(k1, q1, v1, src, dst, zeros)
    t1, ps1, pq1 = _sig_stats(agg1, s1)
    sums0, sums1, counts = _bn_pool(t1, ps1, pq1, g1, be1, h0, batch3)

    wc = jnp.pad(params['cls_W'], ((0, 0), (0, D - params['cls_W'].shape[1])))
    bc = jnp.pad(params['cls_b'], (0, D - params['cls_b'].shape[0])).reshape(1, D)
    out = _head(sums0, sums1, counts,
                params['hl0_W'], params['hl0_b'].reshape(1, -1),
                params['hbn0_gamma'].reshape(1, -1), params['hbn0_beta'].reshape(1, -1),
                params['hl1_W'], params['hl1_b'].reshape(1, -1),
                params['hbn1_gamma'].reshape(1, -1), params['hbn1_beta'].reshape(1, -1),
                wc, bc)
    return out[:, :params['cls_W'].shape[1]]


# speedup vs baseline: 2.6577x; 1.0022x over previous
import functools

import jax
import jax.numpy as jnp
from jax import lax
from jax.experimental import pallas as pl
from jax.experimental.pallas import tpu as pltpu
from jax.experimental.pallas import tpu_sc as plsc

NC = 2
NS = 16
LANES = 16
NW = NC * NS
EPS = 1e-5
BLK = 1000
G = 64


def _sigmoid(z):
    return 1.0 / (1.0 + jnp.exp(-z))


def _edge_phase(k, q, v, src, dst, zeros):
    N, D = k.shape
    E = src.shape[0]
    per_w = E // NW
    C = 40
    chunks = per_w // C
    pairs = chunks // 2
    rem = chunks % 2
    assert chunks >= 4
    rows_pt = 8 * (N // (8 * NS))
    tail_base = NS * rows_pt
    tail_rows = N - tail_base

    mesh = plsc.VectorSubcoreMesh(core_axis_name="c", subcore_axis_name="s",
                                  num_cores=NC, num_subcores=NS)

    @functools.partial(
        pl.kernel,
        out_type=jax.ShapeDtypeStruct((NC, N, D), jnp.float32),
        mesh=mesh,
        scratch_types=[
            pltpu.VMEM((C,), jnp.int32),
            pltpu.VMEM((C,), jnp.int32),
            pltpu.VMEM((C,), jnp.int32),
            pltpu.VMEM((C,), jnp.int32),
            pltpu.VMEM((C,), jnp.int32),
            pltpu.VMEM((C,), jnp.int32),
            pltpu.VMEM((C,), jnp.int32),
            pltpu.VMEM((C,), jnp.int32),
            pltpu.VMEM((2, C, D), jnp.float32),
            pltpu.VMEM((2, C, D), jnp.float32),
            pltpu.VMEM((2, C, D), jnp.float32),
            pltpu.VMEM_SHARED((N, D), jnp.float32),
            pltpu.SemaphoreType.DMA,
            pltpu.SemaphoreType.DMA,
            pltpu.SemaphoreType.DMA,
            pltpu.SemaphoreType.DMA,
            pltpu.SemaphoreType.DMA,
            pltpu.SemaphoreType.DMA,
        ],
    )
    def edge_kernel(k_hbm, q_hbm, v_hbm, src_hbm, dst_hbm, z_hbm, out_hbm,
                    sig0, sig1, sig2, sig3, dig0, dig1, dig2, dig3,
                    kd, qs, vs, acc,
                    isem0, isem1, isem2, isem3, gsem0, gsem1):
        cid = lax.axis_index("c")
        sid = lax.axis_index("s")
        wid = sid * NC + cid
        base0 = wid * per_w
        r0 = sid * rows_pt
        sigs = (sig0, sig1, sig2, sig3)
        digs = (dig0, dig1, dig2, dig3)
        isems = (isem0, isem1, isem2, isem3)
        gsems = (gsem0, gsem1)

        pltpu.sync_copy(z_hbm.at[pl.ds(r0, rows_pt)], acc.at[pl.ds(r0, rows_pt)])
        if tail_rows:
            @pl.when(sid == NS - 1)
            def _():
                pltpu.sync_copy(z_hbm.at[pl.ds(tail_base, tail_rows)],
                                acc.at[pl.ds(tail_base, tail_rows)])

        def fire_idx(j, c):
            pltpu.async_copy(src_hbm.at[pl.ds(base0 + c * C, C)], sigs[j],
                             isems[j])
            pltpu.async_copy(dst_hbm.at[pl.ds(base0 + c * C, C)], digs[j],
                             isems[j])

        def wait_idx(j):
            pltpu.make_async_copy(src_hbm.at[pl.ds(base0, C)], sigs[j],
                                  isems[j]).wait()
            pltpu.make_async_copy(dst_hbm.at[pl.ds(base0, C)], digs[j],
                                  isems[j]).wait()

        def fire_gathers(b, j):
            pltpu.async_copy(k_hbm.at[digs[j]], kd.at[b], gsems[b])
            pltpu.async_copy(q_hbm.at[sigs[j]], qs.at[b], gsems[b])
            pltpu.async_copy(v_hbm.at[sigs[j]], vs.at[b], gsems[b])

        def wait_gathers(b, j):
            pltpu.make_async_copy(k_hbm.at[digs[j]], kd.at[b], gsems[b]).wait()
            pltpu.make_async_copy(q_hbm.at[sigs[j]], qs.at[b], gsems[b]).wait()
            pltpu.make_async_copy(v_hbm.at[sigs[j]], vs.at[b], gsems[b]).wait()

        def compute(b):
            def row_body(r2, carry):
                for u in range(2):
                    r = 2 * r2 + u
                    for cc in range(D // LANES):
                        sl = pl.ds(cc * LANES, LANES)
                        z = kd[b, r, sl] + qs[b, r, sl]
                        vs[b, r, sl] = _sigmoid(z) * vs[b, r, sl]
                return carry
            lax.fori_loop(0, C // 2, row_body, 0)

        def scatter(b, j):
            pltpu.sync_copy(vs.at[b], acc.at[digs[j]], add=True)

        plsc.subcore_barrier()

        for j in range(4):
            fire_idx(j, j)
        wait_idx(0)
        fire_gathers(0, 0)
        wait_idx(1)
        fire_gathers(1, 1)

        def body(c, b, j, jn):
            wait_gathers(b, j)
            compute(b)
            scatter(b, j)

            @pl.when(c + 2 < chunks)
            def _():
                wait_idx(jn)
                fire_gathers(b, jn)

            @pl.when(c + 4 < chunks)
            def _():
                fire_idx(j, c + 4)

        quads = chunks // 4
        qrem = chunks % 4

        def quad_body(iq, carry):
            c0 = 4 * iq
            for u in range(4):
                body(c0 + u, u % 2, u, (u + 2) % 4)
            return carry

        lax.fori_loop(0, quads, quad_body, 0)
        for u in range(qrem):
            body(quads * 4 + u, u % 2, u, (u + 2) % 4)

        plsc.subcore_barrier()
        pltpu.sync_copy(acc.at[pl.ds(r0, rows_pt)],
                        out_hbm.at[cid, pl.ds(r0, rows_pt)])
        if tail_rows:
            @pl.when(sid == NS - 1)
            def _():
                pltpu.sync_copy(acc.at[pl.ds(tail_base, tail_rows)],
                                out_hbm.at[cid, pl.ds(tail_base, tail_rows)])

    return edge_kernel(k, q, v, src, dst, zeros)


def _dense4(h, wall, ball):
    N, D = h.shape
    grid = N // BLK

    def kern(h_ref, w_ref, b_ref, k_ref, q_ref, v_ref, s_ref):
        y = jnp.dot(h_ref[...], w_ref[...],
                    preferred_element_type=jnp.float32) + b_ref[...]
        k_ref[...] = y[:, 0 * D:1 * D]
        q_ref[...] = y[:, 1 * D:2 * D]
        v_ref[...] = y[:, 2 * D:3 * D]
        s_ref[...] = y[:, 3 * D:4 * D]

    return pl.pallas_call(
        kern,
        grid=(grid,),
        in_specs=[pl.BlockSpec((BLK, D), lambda i: (i, 0)),
                  pl.BlockSpec((D, 4 * D), lambda i: (0, 0)),
                  pl.BlockSpec((1, 4 * D), lambda i: (0, 0))],
        out_specs=[pl.BlockSpec((BLK, D), lambda i: (i, 0))] * 4,
        out_shape=[jax.ShapeDtypeStruct((N, D), jnp.float32)] * 4,
    )(h, wall, ball)


def _sig_stats(agg2, s):
    N, D = s.shape
    grid = N // BLK

    def kern(a_ref, s_ref, t_ref, ps_ref, pq_ref):
        t = _sigmoid(a_ref[0] + a_ref[1] + s_ref[...])
        t_ref[...] = t
        ps_ref[0, 0, :] = jnp.sum(t, axis=0)
        pq_ref[0, 0, :] = jnp.sum(t * t, axis=0)

    return pl.pallas_call(
        kern,
        grid=(grid,),
        in_specs=[pl.BlockSpec((2, BLK, D), lambda i: (0, i, 0)),
                  pl.BlockSpec((BLK, D), lambda i: (i, 0))],
        out_specs=[pl.BlockSpec((BLK, D), lambda i: (i, 0)),
                   pl.BlockSpec((1, 1, D), lambda i: (i, 0, 0)),
                   pl.BlockSpec((1, 1, D), lambda i: (i, 0, 0))],
        out_shape=[jax.ShapeDtypeStruct((N, D), jnp.float32),
                   jax.ShapeDtypeStruct((grid, 1, D), jnp.float32),
                   jax.ShapeDtypeStruct((grid, 1, D), jnp.float32)],
    )(agg2, s)


def _bn_dense4(t, ps, pq, gamma, beta, wall, ball):
    N, D = t.shape
    grid = N // BLK

    def kern(t_ref, ps_ref, pq_ref, g_ref, be_ref, w_ref, b_ref,
             h_ref, k_ref, q_ref, v_ref, s_ref):
        mu = jnp.sum(ps_ref[...], axis=0) / N
        var = jnp.sum(pq_ref[...], axis=0) / N - mu * mu
        scale = g_ref[...] * lax.rsqrt(var + EPS)
        h = (t_ref[...] - mu) * scale + be_ref[...]
        h_ref[...] = h
        y = jnp.dot(h, w_ref[...], preferred_element_type=jnp.float32) + b_ref[...]
        k_ref[...] = y[:, 0 * D:1 * D]
        q_ref[...] = y[:, 1 * D:2 * D]
        v_ref[...] = y[:, 2 * D:3 * D]
        s_ref[...] = y[:, 3 * D:4 * D]

    return pl.pallas_call(
        kern,
        grid=(grid,),
        in_specs=[pl.BlockSpec((BLK, D), lambda i: (i, 0)),
                  pl.BlockSpec((grid, 1, D), lambda i: (0, 0, 0)),
                  pl.BlockSpec((grid, 1, D), lambda i: (0, 0, 0)),
                  pl.BlockSpec((1, D), lambda i: (0, 0)),
                  pl.BlockSpec((1, D), lambda i: (0, 0)),
                  pl.BlockSpec((D, 4 * D), lambda i: (0, 0)),
                  pl.BlockSpec((1, 4 * D), lambda i: (0, 0))],
        out_specs=[pl.BlockSpec((BLK, D), lambda i: (i, 0))] * 5,
        out_shape=[jax.ShapeDtypeStruct((N, D), jnp.float32)] * 5,
    )(t, ps, pq, gamma, beta, wall, ball)


def _bn_pool(t1, ps, pq, gamma, beta, h0, batch3):
    N, D = t1.shape
    grid = N // BLK

    def kern(t_ref, ps_ref, pq_ref, g_ref, be_ref, h0_ref, bt_ref,
             s0_ref, s1_ref, c_ref):
        i = pl.program_id(0)
        mu = jnp.sum(ps_ref[...], axis=0) / N
        var = jnp.sum(pq_ref[...], axis=0) / N - mu * mu
        scale = g_ref[...] * lax.rsqrt(var + EPS)
        h1 = (t_ref[...] - mu) * scale + be_ref[...]
        gids = lax.broadcasted_iota(jnp.int32, (G, BLK), 0)
        oh = (gids == bt_ref[0]).astype(jnp.float32)
        part0 = jnp.dot(oh, h0_ref[...], preferred_element_type=jnp.float32)
        part1 = jnp.dot(oh, h1, preferred_element_type=jnp.float32)
        cnt = jnp.broadcast_to(jnp.sum(oh, axis=1, keepdims=True), (G, D))

        @pl.when(i == 0)
        def _():
            s0_ref[...] = jnp.zeros((G, D), jnp.float32)
            s1_ref[...] = jnp.zeros((G, D), jnp.float32)
            c_ref[...] = jnp.zeros((G, D), jnp.float32)

        s0_ref[...] += part0
        s1_ref[...] += part1
        c_ref[...] += cnt

    return pl.pallas_call(
        kern,
        grid=(grid,),
        in_specs=[pl.BlockSpec((BLK, D), lambda i: (i, 0)),
                  pl.BlockSpec((grid, 1, D), lambda i: (0, 0, 0)),
                  pl.BlockSpec((grid, 1, D), lambda i: (0, 0, 0)),
                  pl.BlockSpec((1, D), lambda i: (0, 0)),
                  pl.BlockSpec((1, D), lambda i: (0, 0)),
                  pl.BlockSpec((BLK, D), lambda i: (i, 0)),
                  pl.BlockSpec((1, 1, BLK), lambda i: (i, 0, 0))],
        out_specs=[pl.BlockSpec((G, D), lambda i: (0, 0))] * 3,
        out_shape=[jax.ShapeDtypeStruct((G, D), jnp.float32)] * 3,
    )(t1, ps, pq, gamma, beta, h0, batch3)


def _head(s0, s1, cnt, w0, b0, g0, be0, w1, b1, g1, be1, wc, bc):
    def bn(xv, g, b):
        mu = jnp.mean(xv, axis=0, keepdims=True)
        var = jnp.mean((xv - mu) * (xv - mu), axis=0, keepdims=True)
        return (xv - mu) * lax.rsqrt(var + EPS) * g + b

    def kern(s0_ref, s1_ref, c_ref, w0_ref, b0_ref, g0_ref, be0_ref,
             w1_ref, b1_ref, g1_ref, be1_ref, wc_ref, bc_ref, o_ref):
        c = jnp.maximum(c_ref[...], 1.0)
        feat = jnp.concatenate(
            [s0_ref[...] / c, s1_ref[...] / c, s0_ref[...], s1_ref[...]],
            axis=1)
        xv = jnp.dot(feat, w0_ref[...], preferred_element_type=jnp.float32) + b0_ref[...]
        xv = bn(_sigmoid(xv), g0_ref[...], be0_ref[...])
        xv = jnp.dot(xv, w1_ref[...], preferred_element_type=jnp.float32) + b1_ref[...]
        xv = bn(_sigmoid(xv), g1_ref[...], be1_ref[...])
        o_ref[...] = jnp.dot(xv, wc_ref[...], preferred_element_type=jnp.float32) + bc_ref[...]

    D = s0.shape[1]
    return pl.pallas_call(
        kern,
        out_shape=jax.ShapeDtypeStruct((G, D), jnp.float32),
    )(s0, s1, cnt, w0, b0, g0, be0, w1, b1, g1, be1, wc, bc)


def kernel(x, edge_index, batch, params):
    N, D = x.shape
    src = edge_index[0]
    dst = edge_index[1]
    grid = N // BLK
    zeros = jnp.zeros((N, D), jnp.float32)
    batch3 = batch.reshape(grid, 1, BLK)

    def wall(i):
        w = jnp.concatenate([params['conv%d_%s_W' % (i, nm)]
                             for nm in ('key', 'query', 'value', 'skip')], axis=1)
        b = jnp.concatenate([params['conv%d_%s_b' % (i, nm)]
                             for nm in ('key', 'query', 'value', 'skip')]).reshape(1, 4 * D)
        return w, b

    w0c, b0c = wall(0)
    w1c, b1c = wall(1)
    g0 = params['bn0_gamma'].reshape(1, D)
    be0 = params['bn0_beta'].reshape(1, D)
    g1 = params['bn1_gamma'].reshape(1, D)
    be1 = params['bn1_beta'].reshape(1, D)

    k0, q0, v0, s0 = _dense4(x, w0c, b0c)
    agg0 = _edge_phase(k0, q0, v0, src, dst, zeros)
    t0, ps0, pq0 = _sig_stats(agg0, s0)
    h0, k1, q1, v1, s1 = _bn_dense4(t0, ps0, pq0, g0, be0, w1c, b1c)
    agg1 = _edge_phase(k1, q1, v1, src, dst, zeros)
    t1, ps1, pq1 = _sig_stats(agg1, s1)
    sums0, sums1, counts = _bn_pool(t1, ps1, pq1, g1, be1, h0, batch3)

    wc = jnp.pad(params['cls_W'], ((0, 0), (0, D - params['cls_W'].shape[1])))
    bc = jnp.pad(params['cls_b'], (0, D - params['cls_b'].shape[0])).reshape(1, D)
    out = _head(sums0, sums1, counts,
                params['hl0_W'], params['hl0_b'].reshape(1, -1),
                params['hbn0_gamma'].reshape(1, -1), params['hbn0_beta'].reshape(1, -1),
                params['hl1_W'], params['hl1_b'].reshape(1, -1),
                params['hbn1_gamma'].reshape(1, -1), params['hbn1_beta'].reshape(1, -1),
                wc, bc)
    return out[:, :params['cls_W'].shape[1]]
